# baseline (speedup 1.0000x reference)
import functools

import jax
import jax.numpy as jnp
from jax import lax
from jax.experimental import pallas as pl
from jax.experimental.pallas import tpu as pltpu
import jax.experimental.pallas.tpu_sc as plsc

_N = 10000
_NP = 10240
_E = 320000
_EP = 327680
_K1 = 128
_K2 = 64
_GRP = 16
_RPT = _NP // 16
_F1 = 72
_F2 = 144
_H = 256
_C = 40
_MESH = plsc.VectorSubcoreMesh(core_axis_name="c", subcore_axis_name="s")
_SC_PARAMS = pltpu.CompilerParams(use_tc_tiling_on_sc=False)


def _conv1_body(x01, nidr, srcr, dstr, dstN, z72, z8, ones8,
                o_acc, o_cnt,
                xsh, acc_sh, cnt_sh, nid2, rows_v, rows_b, rows8,
                sidx2, didx2, didxB, ones_v, sems):
    cid = lax.axis_index("c")
    sid = lax.axis_index("s")
    base = sid * _RPT
    nsub = _RPT // _K1

    pltpu.sync_copy(z72, rows_v)
    pltpu.sync_copy(z8, rows8)
    pltpu.sync_copy(ones8, ones_v)
    for i in range(nsub):
        pltpu.sync_copy(rows_v, acc_sh.at[pl.ds(base + i * _K1, _K1)])
        pltpu.sync_copy(rows8, cnt_sh.at[pl.ds(base + i * _K1, _K1)])

    pltpu.sync_copy(nidr.at[cid * 16 + sid], nid2)
    bufs = (rows_v, rows_b)
    pltpu.async_copy(x01.at[nid2.at[0]], bufs[0], sems.at[0])
    for i in range(nsub):
        p = i & 1
        if i + 1 < nsub:
            pltpu.async_copy(x01.at[nid2.at[i + 1]], bufs[1 - p],
                             sems.at[1 - p])
        pltpu.make_async_copy(x01.at[nid2.at[i]], bufs[p], sems.at[p]).wait()
        pltpu.sync_copy(bufs[p], xsh.at[pl.ds(base + i * _K1, _K1)])

    plsc.subcore_barrier()

    gsem = (sems.at[0], sems.at[1])
    ssem = (sems.at[2], sems.at[3])
    n_groups = _EP // (16 * _K1 * _GRP)
    naux = n_groups // 2

    def group_body(g, carry):
        pltpu.sync_copy(srcr.at[sid * n_groups + g], sidx2)
        pltpu.sync_copy(dstr.at[sid * n_groups + g], didx2)

        @pl.when(g < naux)
        def _():
            pltpu.sync_copy(dstN.at[(cid * 16 + sid) * naux + g], didxB)

        pltpu.async_copy(xsh.at[sidx2.at[0]], bufs[0], gsem[0])

        sdesc = [None, None]
        odesc = [None]
        for j in range(_GRP):
            p = j & 1
            q = 1 - p
            idxs = sidx2.at[j]
            idxd = didx2.at[j]
            if j + 1 < _GRP:
                if j >= 1:
                    sdesc[q].wait()
                pltpu.async_copy(xsh.at[sidx2.at[j + 1]], bufs[q], gsem[q])
            pltpu.make_async_copy(xsh.at[idxs], bufs[p], gsem[p]).wait()
            sdesc[p] = pltpu.async_copy(bufs[p], acc_sh.at[idxd], ssem[p],
                                        add=True)

            @pl.when(g < naux)
            def _():
                if odesc[0] is not None:
                    odesc[0].wait()
                odesc[0] = pltpu.async_copy(ones_v, cnt_sh.at[didxB.at[j]],
                                            sems.at[4], add=True)

        sdesc[_GRP & 1].wait()
        sdesc[(_GRP - 1) & 1].wait()

        @pl.when(g < naux)
        def _():
            odesc[0].wait()

        return carry

    lax.fori_loop(0, n_groups, group_body, 0)

    plsc.subcore_barrier()

    for i in range(nsub):
        r0 = base + i * _K1
        pltpu.sync_copy(acc_sh.at[pl.ds(r0, _K1)], rows_v)
        pltpu.sync_copy(rows_v, o_acc.at[pl.ds(cid * _NP + r0, _K1)])
        pltpu.sync_copy(cnt_sh.at[pl.ds(r0, _K1)], rows8)
        pltpu.sync_copy(rows8, o_cnt.at[pl.ds(cid * _NP + r0, _K1)])


_conv1 = functools.partial(
    pl.kernel,
    out_type=(
        jax.ShapeDtypeStruct((2 * _NP, _F1), jnp.float32),
        jax.ShapeDtypeStruct((2 * _NP, 8), jnp.float32),
    ),
    mesh=_MESH,
    scratch_types=[
        pltpu.VMEM_SHARED((_NP, _F1), jnp.float32),
        pltpu.VMEM_SHARED((_NP, _F1), jnp.float32),
        pltpu.VMEM_SHARED((_NP, 8), jnp.float32),
        pltpu.VMEM((_RPT // _K1, _K1), jnp.int32),
        pltpu.VMEM((_K1, _F1), jnp.float32),
        pltpu.VMEM((_K1, _F1), jnp.float32),
        pltpu.VMEM((_K1, 8), jnp.float32),
        pltpu.VMEM((_GRP, _K1), jnp.int32),
        pltpu.VMEM((_GRP, _K1), jnp.int32),
        pltpu.VMEM((_GRP, _K1), jnp.int32),
        pltpu.VMEM((_K1, 8), jnp.float32),
        pltpu.SemaphoreType.DMA((5,)),
    ],
    compiler_params=_SC_PARAMS,
)(_conv1_body)


def _conv2_body(h1f, srcb, dstr, z128,
                o_acc,
                acc_sh, rows_v, rows_b, sidx2, didx2, sems):
    cid = lax.axis_index("c")
    sid = lax.axis_index("s")
    base = sid * _RPT
    nsub = _RPT // _K1

    pltpu.sync_copy(z128, rows_v)
    for i in range(nsub):
        pltpu.sync_copy(rows_v, acc_sh.at[pl.ds(base + i * _K1, _K1)])
    plsc.subcore_barrier()

    gsem = (sems.at[0], sems.at[1])
    ssem = (sems.at[2], sems.at[3])
    n_groups = _EP // (16 * _K1 * _GRP)

    def group_body(g, carry):
        pltpu.sync_copy(srcb.at[(cid * 16 + sid) * n_groups + g], sidx2)
        pltpu.sync_copy(dstr.at[sid * n_groups + g], didx2)

        bufs = (rows_v, rows_b)
        pltpu.async_copy(h1f.at[sidx2.at[0]], bufs[0], gsem[0])

        sdesc = [None, None]
        for j in range(_GRP):
            p = j & 1
            q = 1 - p
            idxs = sidx2.at[j]
            idxd = didx2.at[j]
            if j + 1 < _GRP:
                if j >= 1:
                    sdesc[q].wait()
                pltpu.async_copy(h1f.at[sidx2.at[j + 1]], bufs[q], gsem[q])
            pltpu.make_async_copy(h1f.at[idxs], bufs[p], gsem[p]).wait()
            sdesc[p] = pltpu.async_copy(bufs[p], acc_sh.at[idxd], ssem[p],
                                        add=True)

        sdesc[_GRP & 1].wait()
        sdesc[(_GRP - 1) & 1].wait()
        return carry

    lax.fori_loop(0, n_groups, group_body, 0)

    plsc.subcore_barrier()

    for i in range(nsub):
        r0 = base + i * _K1
        pltpu.sync_copy(acc_sh.at[pl.ds(r0, _K1)], rows_v)
        pltpu.sync_copy(rows_v, o_acc.at[pl.ds(cid * _NP + r0, _K1)])


_conv2 = functools.partial(
    pl.kernel,
    out_type=jax.ShapeDtypeStruct((2 * _NP, 128), jnp.float32),
    mesh=_MESH,
    scratch_types=[
        pltpu.VMEM_SHARED((_NP, 128), jnp.float32),
        pltpu.VMEM((_K1, 128), jnp.float32),
        pltpu.VMEM((_K1, 128), jnp.float32),
        pltpu.VMEM((_GRP, _K1), jnp.int32),
        pltpu.VMEM((_GRP, _K1), jnp.int32),
        pltpu.SemaphoreType.DMA((4,)),
    ],
    compiler_params=pltpu.CompilerParams(use_tc_tiling_on_sc=True),
)(_conv2_body)


def _mm1_body(agg_ref, w_ref, b_ref, o_ref):
    a0 = agg_ref[0]
    a1 = agg_ref[1]
    cnt = a0[:, 64:65]
    inv = 1.0 / jnp.maximum(cnt, 1.0)
    x = jnp.concatenate([a1[:, 56:64], a0[:, 0:64], a1[:, 0:56]], axis=1) * inv
    h = jnp.dot(x, w_ref[...], preferred_element_type=jnp.float32) + b_ref[...]
    h = jnp.maximum(h, 0.0)
    o_ref[0] = h[:, 0:128]
    o_ref[1] = h[:, 128:256]


def _mm2_body(agg_ref, cnt_ref, w2_ref, b2_ref, wo_ref, bo_ref, o_ref):
    cnt = cnt_ref[0, :, 0:1] + cnt_ref[1, :, 0:1]
    inv = 1.0 / jnp.maximum(cnt, 1.0)
    x = jnp.concatenate([agg_ref[0], agg_ref[1]], axis=1) * inv
    h = jnp.dot(x, w2_ref[...], preferred_element_type=jnp.float32) + b2_ref[...]
    h = jnp.maximum(h, 0.0)
    lg = jnp.dot(h, wo_ref[...], preferred_element_type=jnp.float32) + bo_ref[...]
    m = jnp.max(lg, axis=1, keepdims=True)
    e = jnp.exp(lg - m)
    s = jnp.sum(e, axis=1, keepdims=True)
    o_ref[...] = lg - m - jnp.log(s)


_BN = 1000


def _mm1(agg, w1, b1):
    return pl.pallas_call(
        _mm1_body,
        grid=(_N // _BN,),
        in_specs=[
            pl.BlockSpec((2, _BN, _F1), lambda i: (0, i, 0)),
            pl.BlockSpec((128, _H), lambda i: (0, 0)),
            pl.BlockSpec((1, _H), lambda i: (0, 0)),
        ],
        out_specs=pl.BlockSpec((2, _BN, 128), lambda i: (0, i, 0)),
        out_shape=jax.ShapeDtypeStruct((2, _N, 128), jnp.float32),
    )(agg, w1, b1)


def _mm2(agg, cnt, w2, b2, wo, bo):
    return pl.pallas_call(
        _mm2_body,
        grid=(_N // _BN,),
        in_specs=[
            pl.BlockSpec((2, _BN, 128), lambda i: (0, i, 0)),
            pl.BlockSpec((2, _BN, 8), lambda i: (0, i, 0)),
            pl.BlockSpec((_H, _H), lambda i: (0, 0)),
            pl.BlockSpec((1, _H), lambda i: (0, 0)),
            pl.BlockSpec((_H, _C), lambda i: (0, 0)),
            pl.BlockSpec((1, _C), lambda i: (0, 0)),
        ],
        out_specs=pl.BlockSpec((_BN, _C), lambda i: (i, 0)),
        out_shape=jax.ShapeDtypeStruct((_N, _C), jnp.float32),
    )(agg, cnt, w2, b2, wo, bo)


def kernel(data_x, data_node_index, data_node_one_hot, n_id0, edge_index0,
           n_id1, edge_index1, emb_table, W1, b1, W2, b2, Wout, bout):
    del data_node_index, data_node_one_hot, n_id1
    ones_n = jnp.ones((_N, 8), jnp.float32)
    zeros_n = jnp.zeros((_N, 8), jnp.float32)
    x0a = jnp.concatenate([data_x[:, 0:64], ones_n], axis=1)
    x1a = jnp.concatenate([data_x[:, 64:120], emb_table, zeros_n], axis=1)
    x01 = jnp.concatenate([x0a, x1a], axis=0)

    nid_pad = jnp.concatenate([n_id0, jnp.zeros((_NP - _N,), jnp.int32)])
    nidr = jnp.concatenate([nid_pad, nid_pad + _N]).reshape(32, -1, _K1)

    pidx = jnp.arange(_EP - _E, dtype=jnp.int32)
    ps = pidx % _N
    pd = _N + pidx % (_NP - _N)
    src0r = jnp.concatenate([edge_index0[0], ps]).reshape(-1, _GRP, _K1)
    dst0r = jnp.concatenate([edge_index0[1], pd]).reshape(-1, _GRP, _K1)
    s1p = jnp.concatenate([edge_index1[0], ps])
    srcb = jnp.concatenate([s1p, s1p + _N]).reshape(-1, _GRP, _K1)
    dst1r = jnp.concatenate([edge_index1[1], pd]).reshape(-1, _GRP, _K1)

    z72 = jnp.zeros((_K1, _F1), jnp.float32)
    z128 = jnp.zeros((_K1, 128), jnp.float32)
    z8 = jnp.zeros((_K1, 8), jnp.float32)
    ones8 = jnp.ones((_K1, 8), jnp.float32)

    o1, o_cnt = _conv1(x01, nidr, src0r, dst0r, dst1r, z72, z8, ones8)
    h1s = _mm1(o1.reshape(2, _NP, _F1), W1, b1.reshape(1, _H))
    o3 = _conv2(h1s.reshape(2 * _N, 128), srcb, dst1r, z128)
    return _mm2(o3.reshape(2, _NP, 128), o_cnt.reshape(2, _NP, 8),
                W2, b2.reshape(1, _H), Wout, bout.reshape(1, _C))


# speedup vs baseline: 1.0120x; 1.0120x over previous
import functools

import jax
import jax.numpy as jnp
from jax import lax
from jax.experimental import pallas as pl
from jax.experimental.pallas import tpu as pltpu
import jax.experimental.pallas.tpu_sc as plsc

_N = 10000
_NP = 10240
_E = 320000
_EP = 327680
_K1 = 128
_K2 = 64
_GRP = 16
_RPT = _NP // 16
_F1 = 72
_F2 = 144
_H = 256
_C = 40
_MESH = plsc.VectorSubcoreMesh(core_axis_name="c", subcore_axis_name="s")
_SC_PARAMS = pltpu.CompilerParams(use_tc_tiling_on_sc=False)


def _conv1_body(x01, emb, nidr, srcr, dstr, z64, z8, ones8,
                o_acc, o_aux,
                xsh, acc_sh, esh, aacc_sh, nid2, rows_v, rows_b,
                rows8, rows8_b, sidx2, didx2, ones_v, sems):
    cid = lax.axis_index("c")
    sid = lax.axis_index("s")
    base = sid * _RPT
    nsub = _RPT // _K1

    pltpu.sync_copy(z64, rows_v)
    pltpu.sync_copy(z8, rows8)
    pltpu.sync_copy(ones8, ones_v)
    for i in range(nsub):
        pltpu.sync_copy(rows_v, acc_sh.at[pl.ds(base + i * _K1, _K1)])
        pltpu.sync_copy(rows8, aacc_sh.at[pl.ds(base + i * _K1, _K1)])

    pltpu.sync_copy(nidr.at[cid * 16 + sid], nid2)
    bufs = (rows_v, rows_b)
    b8s = (rows8, rows8_b)
    pltpu.async_copy(x01.at[nid2.at[0]], bufs[0], sems.at[0])

    @pl.when(cid == 1)
    def _():
        pltpu.async_copy(emb.at[nid2.at[0]], b8s[0], sems.at[4])

    for i in range(nsub):
        p = i & 1
        if i + 1 < nsub:
            pltpu.async_copy(x01.at[nid2.at[i + 1]], bufs[1 - p],
                             sems.at[1 - p])

            @pl.when(cid == 1)
            def _():
                pltpu.async_copy(emb.at[nid2.at[i + 1]], b8s[1 - p],
                                 sems.at[4 + (1 - p)])

        pltpu.make_async_copy(x01.at[nid2.at[i]], bufs[p], sems.at[p]).wait()
        pltpu.sync_copy(bufs[p], xsh.at[pl.ds(base + i * _K1, _K1)])

        @pl.when(cid == 1)
        def _():
            pltpu.make_async_copy(emb.at[nid2.at[i]], b8s[p],
                                  sems.at[4 + p]).wait()
            pltpu.sync_copy(b8s[p], esh.at[pl.ds(base + i * _K1, _K1)])

    plsc.subcore_barrier()

    gsem = (sems.at[0], sems.at[1])
    ssem = (sems.at[2], sems.at[3])
    g8sem = (sems.at[4], sems.at[5])
    s8sem = (sems.at[6], sems.at[7])
    osem = sems.at[8]
    n_groups = _EP // (16 * _K1 * _GRP)

    def group_body(g, carry):
        pltpu.sync_copy(srcr.at[sid * n_groups + g], sidx2)
        pltpu.sync_copy(dstr.at[sid * n_groups + g], didx2)

        pltpu.async_copy(xsh.at[sidx2.at[0]], bufs[0], gsem[0])

        @pl.when(cid == 1)
        def _():
            pltpu.async_copy(esh.at[sidx2.at[0]], b8s[0], g8sem[0])

        sdesc = [None, None]
        edesc = [None, None]
        odesc = [None]
        for j in range(_GRP):
            p = j & 1
            q = 1 - p
            idxs = sidx2.at[j]
            idxd = didx2.at[j]
            if j + 1 < _GRP:
                if j >= 1:
                    sdesc[q].wait()

                    @pl.when(cid == 1)
                    def _():
                        edesc[q].wait()

                idxn = sidx2.at[j + 1]
                pltpu.async_copy(xsh.at[idxn], bufs[q], gsem[q])

                @pl.when(cid == 1)
                def _():
                    pltpu.async_copy(esh.at[idxn], b8s[q], g8sem[q])

            pltpu.make_async_copy(xsh.at[idxs], bufs[p], gsem[p]).wait()
            sdesc[p] = pltpu.async_copy(bufs[p], acc_sh.at[idxd], ssem[p],
                                        add=True)

            @pl.when(cid == 1)
            def _():
                pltpu.make_async_copy(esh.at[idxs], b8s[p], g8sem[p]).wait()
                edesc[p] = pltpu.async_copy(b8s[p], aacc_sh.at[idxd],
                                            s8sem[p], add=True)

            @pl.when(cid == 0)
            def _():
                if odesc[0] is not None:
                    odesc[0].wait()
                odesc[0] = pltpu.async_copy(ones_v, aacc_sh.at[idxd], osem,
                                            add=True)

        for j in (_GRP - 2, _GRP - 1):
            sdesc[j & 1].wait()

            @pl.when(cid == 1)
            def _():
                edesc[j & 1].wait()

        @pl.when(cid == 0)
        def _():
            odesc[0].wait()

        return carry

    lax.fori_loop(0, n_groups, group_body, 0)

    plsc.subcore_barrier()

    for i in range(nsub):
        r0 = base + i * _K1
        pltpu.sync_copy(acc_sh.at[pl.ds(r0, _K1)], rows_v)
        pltpu.sync_copy(rows_v, o_acc.at[pl.ds(cid * _NP + r0, _K1)])
        pltpu.sync_copy(aacc_sh.at[pl.ds(r0, _K1)], rows8)
        pltpu.sync_copy(rows8, o_aux.at[pl.ds(cid * _NP + r0, _K1)])


_conv1 = functools.partial(
    pl.kernel,
    out_type=(
        jax.ShapeDtypeStruct((2 * _NP, 64), jnp.float32),
        jax.ShapeDtypeStruct((2 * _NP, 8), jnp.float32),
    ),
    mesh=_MESH,
    scratch_types=[
        pltpu.VMEM_SHARED((_NP, 64), jnp.float32),
        pltpu.VMEM_SHARED((_NP, 64), jnp.float32),
        pltpu.VMEM_SHARED((_NP, 8), jnp.float32),
        pltpu.VMEM_SHARED((_NP, 8), jnp.float32),
        pltpu.VMEM((_RPT // _K1, _K1), jnp.int32),
        pltpu.VMEM((_K1, 64), jnp.float32),
        pltpu.VMEM((_K1, 64), jnp.float32),
        pltpu.VMEM((_K1, 8), jnp.float32),
        pltpu.VMEM((_K1, 8), jnp.float32),
        pltpu.VMEM((_GRP, _K1), jnp.int32),
        pltpu.VMEM((_GRP, _K1), jnp.int32),
        pltpu.VMEM((_K1, 8), jnp.float32),
        pltpu.SemaphoreType.DMA((9,)),
    ],
    compiler_params=_SC_PARAMS,
)(_conv1_body)


def _conv2_body(h1f, srcb, dstr, z128, z8, ones8,
                o_acc, o_cnt,
                acc_sh, aacc_sh, rows_v, rows_b, rows8, ones_v,
                sidx2, didx2, sems):
    cid = lax.axis_index("c")
    sid = lax.axis_index("s")
    base = sid * _RPT
    nsub = _RPT // _K1

    pltpu.sync_copy(z128, rows_v)
    pltpu.sync_copy(z8, rows8)
    pltpu.sync_copy(ones8, ones_v)
    for i in range(nsub):
        pltpu.sync_copy(rows_v, acc_sh.at[pl.ds(base + i * _K1, _K1)])
        pltpu.sync_copy(rows8, aacc_sh.at[pl.ds(base + i * _K1, _K1)])
    plsc.subcore_barrier()

    gsem = (sems.at[0], sems.at[1])
    ssem = (sems.at[2], sems.at[3])
    n_groups = _EP // (16 * _K1 * _GRP)

    def group_body(g, carry):
        pltpu.sync_copy(srcb.at[(cid * 16 + sid) * n_groups + g], sidx2)
        pltpu.sync_copy(dstr.at[sid * n_groups + g], didx2)

        bufs = (rows_v, rows_b)
        pltpu.async_copy(h1f.at[sidx2.at[0]], bufs[0], gsem[0])

        sdesc = [None, None]
        odesc = [None]
        for j in range(_GRP):
            p = j & 1
            q = 1 - p
            idxs = sidx2.at[j]
            idxd = didx2.at[j]
            if j + 1 < _GRP:
                if j >= 1:
                    sdesc[q].wait()
                pltpu.async_copy(h1f.at[sidx2.at[j + 1]], bufs[q], gsem[q])
            pltpu.make_async_copy(h1f.at[idxs], bufs[p], gsem[p]).wait()
            sdesc[p] = pltpu.async_copy(bufs[p], acc_sh.at[idxd], ssem[p],
                                        add=True)

            @pl.when(cid == 0)
            def _():
                if odesc[0] is not None:
                    odesc[0].wait()
                odesc[0] = pltpu.async_copy(ones_v, aacc_sh.at[idxd],
                                            sems.at[4], add=True)

        sdesc[_GRP & 1].wait()
        sdesc[(_GRP - 1) & 1].wait()

        @pl.when(cid == 0)
        def _():
            odesc[0].wait()

        return carry

    lax.fori_loop(0, n_groups, group_body, 0)

    plsc.subcore_barrier()

    for i in range(nsub):
        r0 = base + i * _K1
        pltpu.sync_copy(acc_sh.at[pl.ds(r0, _K1)], rows_v)
        pltpu.sync_copy(rows_v, o_acc.at[pl.ds(cid * _NP + r0, _K1)])

        @pl.when(cid == 0)
        def _():
            pltpu.sync_copy(aacc_sh.at[pl.ds(r0, _K1)], rows8)
            pltpu.sync_copy(rows8, o_cnt.at[pl.ds(r0, _K1)])


_conv2 = functools.partial(
    pl.kernel,
    out_type=(
        jax.ShapeDtypeStruct((2 * _NP, 128), jnp.float32),
        jax.ShapeDtypeStruct((_NP, 8), jnp.float32),
    ),
    mesh=_MESH,
    scratch_types=[
        pltpu.VMEM_SHARED((_NP, 128), jnp.float32),
        pltpu.VMEM_SHARED((_NP, 8), jnp.float32),
        pltpu.VMEM((_K1, 128), jnp.float32),
        pltpu.VMEM((_K1, 128), jnp.float32),
        pltpu.VMEM((_K1, 8), jnp.float32),
        pltpu.VMEM((_K1, 8), jnp.float32),
        pltpu.VMEM((_GRP, _K1), jnp.int32),
        pltpu.VMEM((_GRP, _K1), jnp.int32),
        pltpu.SemaphoreType.DMA((5,)),
    ],
    compiler_params=_SC_PARAMS,
)(_conv2_body)


def _mm1_body(agg_ref, aux_ref, w_ref, b_ref, o_ref):
    a0 = agg_ref[0]
    a1 = agg_ref[1]
    ae = aux_ref[1]
    cnt = aux_ref[0, :, 0:1]
    inv = 1.0 / jnp.maximum(cnt, 1.0)
    x = jnp.concatenate([ae, a0, a1[:, 0:56]], axis=1) * inv
    h = jnp.dot(x, w_ref[...], preferred_element_type=jnp.float32) + b_ref[...]
    h = jnp.maximum(h, 0.0)
    o_ref[0] = h[:, 0:128]
    o_ref[1] = h[:, 128:256]


def _mm2_body(agg_ref, cnt_ref, w2_ref, b2_ref, wo_ref, bo_ref, o_ref):
    cnt = cnt_ref[:, 0:1]
    inv = 1.0 / jnp.maximum(cnt, 1.0)
    x = jnp.concatenate([agg_ref[0], agg_ref[1]], axis=1) * inv
    h = jnp.dot(x, w2_ref[...], preferred_element_type=jnp.float32) + b2_ref[...]
    h = jnp.maximum(h, 0.0)
    lg = jnp.dot(h, wo_ref[...], preferred_element_type=jnp.float32) + bo_ref[...]
    m = jnp.max(lg, axis=1, keepdims=True)
    e = jnp.exp(lg - m)
    s = jnp.sum(e, axis=1, keepdims=True)
    o_ref[...] = lg - m - jnp.log(s)


_BN = 1000


def _mm1(agg, aux, w1, b1):
    return pl.pallas_call(
        _mm1_body,
        grid=(_N // _BN,),
        in_specs=[
            pl.BlockSpec((2, _BN, 64), lambda i: (0, i, 0)),
            pl.BlockSpec((2, _BN, 8), lambda i: (0, i, 0)),
            pl.BlockSpec((128, _H), lambda i: (0, 0)),
            pl.BlockSpec((1, _H), lambda i: (0, 0)),
        ],
        out_specs=pl.BlockSpec((2, _BN, 128), lambda i: (0, i, 0)),
        out_shape=jax.ShapeDtypeStruct((2, _N, 128), jnp.float32),
    )(agg, aux, w1, b1)


def _mm2(agg, cnt, w2, b2, wo, bo):
    return pl.pallas_call(
        _mm2_body,
        grid=(_N // _BN,),
        in_specs=[
            pl.BlockSpec((2, _BN, 128), lambda i: (0, i, 0)),
            pl.BlockSpec((_BN, 8), lambda i: (i, 0)),
            pl.BlockSpec((_H, _H), lambda i: (0, 0)),
            pl.BlockSpec((1, _H), lambda i: (0, 0)),
            pl.BlockSpec((_H, _C), lambda i: (0, 0)),
            pl.BlockSpec((1, _C), lambda i: (0, 0)),
        ],
        out_specs=pl.BlockSpec((_BN, _C), lambda i: (i, 0)),
        out_shape=jax.ShapeDtypeStruct((_N, _C), jnp.float32),
    )(agg, cnt, w2, b2, wo, bo)


def kernel(data_x, data_node_index, data_node_one_hot, n_id0, edge_index0,
           n_id1, edge_index1, emb_table, W1, b1, W2, b2, Wout, bout):
    del data_node_index, data_node_one_hot, n_id1
    zeros_n = jnp.zeros((_N, 8), jnp.float32)
    x0a = data_x[:, 0:64]
    x1a = jnp.concatenate([data_x[:, 64:120], zeros_n], axis=1)
    x01 = jnp.concatenate([x0a, x1a], axis=0)

    nid_pad = jnp.concatenate([n_id0, jnp.zeros((_NP - _N,), jnp.int32)])
    nidr = jnp.concatenate([nid_pad, nid_pad + _N]).reshape(32, -1, _K1)

    pidx = jnp.arange(_EP - _E, dtype=jnp.int32)
    ps = pidx % _N
    pd = _N + pidx % (_NP - _N)
    src0r = jnp.concatenate([edge_index0[0], ps]).reshape(-1, _GRP, _K1)
    dst0r = jnp.concatenate([edge_index0[1], pd]).reshape(-1, _GRP, _K1)
    s1p = jnp.concatenate([edge_index1[0], ps])
    srcb = jnp.concatenate([s1p, s1p + _N]).reshape(-1, _GRP, _K1)
    dst1r = jnp.concatenate([edge_index1[1], pd]).reshape(-1, _GRP, _K1)

    z64 = jnp.zeros((_K1, 64), jnp.float32)
    z128 = jnp.zeros((_K1, 128), jnp.float32)
    z8 = jnp.zeros((_K1, 8), jnp.float32)
    ones8 = jnp.ones((_K1, 8), jnp.float32)

    emb2 = jnp.concatenate([emb_table, emb_table], axis=0)
    o1, o2 = _conv1(x01, emb2, nidr, src0r, dst0r, z64, z8, ones8)
    h1s = _mm1(o1.reshape(2, _NP, 64), o2.reshape(2, _NP, 8),
               W1, b1.reshape(1, _H))
    o3, o4 = _conv2(h1s.reshape(2 * _N, 128), srcb, dst1r, z128, z8, ones8)
    return _mm2(o3.reshape(2, _NP, 128), o4, W2, b2.reshape(1, _H),
                Wout, bout.reshape(1, _C))
